# baseline (speedup 1.0000x reference)
import functools

import jax
import jax.numpy as jnp
from jax import lax
from jax.experimental import pallas as pl
from jax.experimental.pallas import tpu as pltpu
from jax.experimental.pallas import tpu_sc as plsc

_TBLK = 256


def _dist_argmin_body(xb_ref, chi_ref, clo_ref, x2_ref, c2_ref,
                      enc_ref, minsum_ref):
    i = pl.program_id(0)
    xb = xb_ref[...]
    dn = (((1,), (0,)), ((), ()))
    mm = lax.dot_general(xb, chi_ref[...], dn,
                         preferred_element_type=jnp.float32)
    mm = mm + lax.dot_general(xb, clo_ref[...], dn,
                              preferred_element_type=jnp.float32)
    d2 = jnp.maximum((x2_ref[...] + c2_ref[...]) - 2.0 * mm, 0.0)
    dist = jnp.where(d2 == 0.0, 0.0, d2 * lax.rsqrt(d2))
    minval = jnp.min(dist, axis=1, keepdims=True)
    v = dist.shape[1]
    idx = lax.broadcasted_iota(jnp.int32, dist.shape, 1)
    enc = jnp.min(jnp.where(dist == minval, idx, v),
                  axis=1, keepdims=True)
    enc_ref[...] = enc

    @pl.when(i == 0)
    def _():
        minsum_ref[...] = jnp.zeros_like(minsum_ref)

    minsum_ref[...] += jnp.sum(jnp.min(d2, axis=1))[None, None]


def _dist_argmin(xb, chit, clot, x2, c2):
    n, d = xb.shape
    v = chit.shape[1]
    grid = n // _TBLK
    return pl.pallas_call(
        _dist_argmin_body,
        grid=(grid,),
        in_specs=[
            pl.BlockSpec((_TBLK, d), lambda i: (i, 0)),
            pl.BlockSpec((d, v), lambda i: (0, 0)),
            pl.BlockSpec((d, v), lambda i: (0, 0)),
            pl.BlockSpec((_TBLK, 1), lambda i: (i, 0)),
            pl.BlockSpec((1, v), lambda i: (0, 0)),
        ],
        out_specs=[
            pl.BlockSpec((_TBLK, 1), lambda i: (i, 0)),
            pl.BlockSpec((1, 1), lambda i: (0, 0)),
        ],
        out_shape=[
            jax.ShapeDtypeStruct((n, 1), jnp.int32),
            jax.ShapeDtypeStruct((1, 1), jnp.float32),
        ],
        compiler_params=pltpu.CompilerParams(
            dimension_semantics=("arbitrary",)),
    )(xb, chit, clot, x2, c2)


def _sc_gather(table, idx):
    v, d = table.shape
    n = idx.shape[0]
    info = plsc.get_sparse_core_info()
    nw = info.num_cores * info.num_subcores
    bpw = n // nw
    mesh = plsc.VectorSubcoreMesh(core_axis_name="c", subcore_axis_name="s")

    @functools.partial(
        pl.kernel,
        mesh=mesh,
        out_type=jax.ShapeDtypeStruct((n, d), jnp.float32),
        scratch_types=[
            pltpu.VMEM((bpw,), jnp.int32),
            pltpu.VMEM((bpw, d), jnp.float32),
            pltpu.SemaphoreType.DMA,
        ],
        compiler_params=pltpu.CompilerParams(use_tc_tiling_on_sc=False),
    )
    def gather(table_hbm, idx_hbm, out_hbm, idx_v, rows_v, sem):
        wid = lax.axis_index("s") * info.num_cores + lax.axis_index("c")
        base = wid * bpw
        pltpu.sync_copy(idx_hbm.at[pl.ds(base, bpw)], idx_v)
        pltpu.async_copy(table_hbm.at[idx_v], rows_v, sem).wait()
        pltpu.sync_copy(rows_v, out_hbm.at[pl.ds(base, bpw)])

    return gather(table, idx)


def kernel(x, codebook):
    b, s, d = x.shape
    xf = x.reshape(-1, d)
    x2 = jnp.sum(xf * xf, axis=1, keepdims=True)
    c2 = jnp.sum(codebook * codebook, axis=1)[None, :]
    xb = xf.astype(jnp.bfloat16)
    c_hi = codebook.astype(jnp.bfloat16)
    c_lo = (codebook - c_hi.astype(jnp.float32)).astype(jnp.bfloat16)
    enc2, minsum = _dist_argmin(xb, c_hi.T.astype(jnp.bfloat16),
                                c_lo.T.astype(jnp.bfloat16), x2, c2)
    enc = enc2.reshape(-1)
    nearest = _sc_gather(codebook, enc)
    loss = minsum[0, 0] / (xf.shape[0] * d)
    return (enc.reshape(b, s), loss, loss, nearest.reshape(b, s, d))
